# speedup vs baseline: 1.2857x; 1.2857x over previous
import jax
import jax.numpy as jnp
from jax import lax
from jax.experimental import pallas as pl
from jax.experimental.pallas import tpu as pltpu
from jax.experimental.pallas import tpu_sc as plsc

BATCH = 4096
SEQ = 200
DIM = 128
NCORES = 2
NSUB = 16
NW = NCORES * NSUB
BPT = BATCH // NW
CH0 = 128
CH1 = SEQ - CH0
NVR = DIM // 16


def _body(instr_hbm, table_hbm, out_hbm, idx_v, rows_v, out_v, sem):
    c = lax.axis_index("c")
    s = lax.axis_index("s")
    wid = s * NCORES + c
    base = wid * BPT

    pltpu.sync_copy(instr_hbm.at[pl.ds(base, BPT), :], idx_v)

    def bag(b, carry):
        cp0 = pltpu.async_copy(
            table_hbm.at[idx_v.at[b, pl.ds(0, CH0)]],
            rows_v.at[pl.ds(0, CH0), :], sem)
        cp1 = pltpu.async_copy(
            table_hbm.at[idx_v.at[b, pl.ds(CH0, CH1)]],
            rows_v.at[pl.ds(CH0, CH1), :], sem)
        cp0.wait()
        cp1.wait()

        def red(r, acc):
            return tuple(acc[d] + rows_v[r, pl.ds(d * 16, 16)]
                         for d in range(NVR))

        acc = lax.fori_loop(
            0, SEQ, red,
            tuple(jnp.zeros((16,), jnp.float32) for _ in range(NVR)))
        for d in range(NVR):
            out_v[b, pl.ds(d * 16, 16)] = acc[d]
        return carry

    lax.fori_loop(0, BPT, bag, 0)
    pltpu.sync_copy(out_v, out_hbm.at[pl.ds(base, BPT), :])


def kernel(instruction, table):
    mesh = plsc.VectorSubcoreMesh(
        core_axis_name="c", subcore_axis_name="s",
        num_cores=NCORES, num_subcores=NSUB)
    run = pl.kernel(
        _body,
        out_type=jax.ShapeDtypeStruct((BATCH, DIM), jnp.float32),
        mesh=mesh,
        scratch_types=[
            pltpu.VMEM((BPT, SEQ), jnp.int32),
            pltpu.VMEM((SEQ, DIM), jnp.float32),
            pltpu.VMEM((BPT, DIM), jnp.float32),
            pltpu.SemaphoreType.DMA,
        ],
    )
    return run(instruction.astype(jnp.int32), table)


# speedup vs baseline: 2.2871x; 1.7788x over previous
import jax
import jax.numpy as jnp
from jax import lax
from jax.experimental import pallas as pl
from jax.experimental.pallas import tpu as pltpu
from jax.experimental.pallas import tpu_sc as plsc

BATCH = 4096
SEQ = 200
DIM = 128
NCORES = 2
NSUB = 16
NW = NCORES * NSUB
BPT = BATCH // NW
CH0 = 128
CH1 = SEQ - CH0
NVR = DIM // 16


UNROLL = 4


def _body(instr_hbm, table_hbm, out_hbm, idx_v, rows_v, out_v, sems):
    c = lax.axis_index("c")
    s = lax.axis_index("s")
    wid = s * NCORES + c
    base = wid * BPT

    pltpu.sync_copy(instr_hbm.at[pl.ds(base, BPT), :], idx_v)

    def start_gather(b, buf):
        pltpu.async_copy(
            table_hbm.at[idx_v.at[b, pl.ds(0, CH0)]],
            rows_v.at[buf, pl.ds(0, CH0), :], sems.at[buf])
        pltpu.async_copy(
            table_hbm.at[idx_v.at[b, pl.ds(CH0, CH1)]],
            rows_v.at[buf, pl.ds(CH0, CH1), :], sems.at[buf])

    def wait_gather(buf):
        pltpu.make_async_copy(
            table_hbm.at[pl.ds(0, CH0), :],
            rows_v.at[buf, pl.ds(0, CH0), :], sems.at[buf]).wait()
        pltpu.make_async_copy(
            table_hbm.at[pl.ds(0, CH1), :],
            rows_v.at[buf, pl.ds(CH0, CH1), :], sems.at[buf]).wait()

    start_gather(0, 0)

    def bag(b, carry):
        p = lax.rem(b, 2)

        @pl.when(b + 1 < BPT)
        def _():
            start_gather(b + 1, 1 - p)

        wait_gather(p)

        def red(r4, acc):
            r = r4 * UNROLL
            out = []
            for d in range(NVR):
                t0 = rows_v[p, r, pl.ds(d * 16, 16)] \
                    + rows_v[p, r + 1, pl.ds(d * 16, 16)]
                t1 = rows_v[p, r + 2, pl.ds(d * 16, 16)] \
                    + rows_v[p, r + 3, pl.ds(d * 16, 16)]
                out.append(acc[d] + (t0 + t1))
            return tuple(out)

        acc = lax.fori_loop(
            0, SEQ // UNROLL, red,
            tuple(jnp.zeros((16,), jnp.float32) for _ in range(NVR)))
        for d in range(NVR):
            out_v[b, pl.ds(d * 16, 16)] = acc[d]
        return carry

    lax.fori_loop(0, BPT, bag, 0)
    pltpu.sync_copy(out_v, out_hbm.at[pl.ds(base, BPT), :])


def kernel(instruction, table):
    mesh = plsc.VectorSubcoreMesh(
        core_axis_name="c", subcore_axis_name="s",
        num_cores=NCORES, num_subcores=NSUB)
    run = pl.kernel(
        _body,
        out_type=jax.ShapeDtypeStruct((BATCH, DIM), jnp.float32),
        mesh=mesh,
        scratch_types=[
            pltpu.VMEM((BPT, SEQ), jnp.int32),
            pltpu.VMEM((2, SEQ, DIM), jnp.float32),
            pltpu.VMEM((BPT, DIM), jnp.float32),
            pltpu.SemaphoreType.DMA((2,)),
        ],
    )
    return run(instruction.astype(jnp.int32), table)


# speedup vs baseline: 2.2967x; 1.0042x over previous
import jax
import jax.numpy as jnp
from jax import lax
from jax.experimental import pallas as pl
from jax.experimental.pallas import tpu as pltpu
from jax.experimental.pallas import tpu_sc as plsc

BATCH = 4096
SEQ = 200
DIM = 128
NCORES = 2
NSUB = 16
NW = NCORES * NSUB
BPT = BATCH // NW
CH0 = 128
CH1 = SEQ - CH0
NVR = DIM // 16


UNROLL = 8


def _body(instr_hbm, table_hbm, out_hbm, idx_v, rows_v, out_v, sems):
    c = lax.axis_index("c")
    s = lax.axis_index("s")
    wid = s * NCORES + c
    base = wid * BPT

    pltpu.sync_copy(instr_hbm.at[pl.ds(base, BPT), :], idx_v)

    def start_gather(b, buf):
        pltpu.async_copy(
            table_hbm.at[idx_v.at[b, pl.ds(0, CH0)]],
            rows_v.at[buf, pl.ds(0, CH0), :], sems.at[buf])
        pltpu.async_copy(
            table_hbm.at[idx_v.at[b, pl.ds(CH0, CH1)]],
            rows_v.at[buf, pl.ds(CH0, CH1), :], sems.at[buf])

    def wait_gather(buf):
        pltpu.make_async_copy(
            table_hbm.at[pl.ds(0, CH0), :],
            rows_v.at[buf, pl.ds(0, CH0), :], sems.at[buf]).wait()
        pltpu.make_async_copy(
            table_hbm.at[pl.ds(0, CH1), :],
            rows_v.at[buf, pl.ds(CH0, CH1), :], sems.at[buf]).wait()

    start_gather(0, 0)

    def bag(b, carry):
        p = lax.rem(b, 2)

        @pl.when(b + 1 < BPT)
        def _():
            start_gather(b + 1, 1 - p)

        wait_gather(p)

        def red(r4, acc):
            r = r4 * UNROLL
            out = []
            for d in range(NVR):
                vals = [rows_v[p, r + k, pl.ds(d * 16, 16)]
                        for k in range(UNROLL)]
                while len(vals) > 1:
                    vals = [vals[i] + vals[i + 1]
                            for i in range(0, len(vals), 2)]
                out.append(acc[d] + vals[0])
            return tuple(out)

        acc = lax.fori_loop(
            0, SEQ // UNROLL, red,
            tuple(jnp.zeros((16,), jnp.float32) for _ in range(NVR)))
        for d in range(NVR):
            out_v[b, pl.ds(d * 16, 16)] = acc[d]
        return carry

    lax.fori_loop(0, BPT, bag, 0)
    pltpu.sync_copy(out_v, out_hbm.at[pl.ds(base, BPT), :])


def kernel(instruction, table):
    mesh = plsc.VectorSubcoreMesh(
        core_axis_name="c", subcore_axis_name="s",
        num_cores=NCORES, num_subcores=NSUB)
    run = pl.kernel(
        _body,
        out_type=jax.ShapeDtypeStruct((BATCH, DIM), jnp.float32),
        mesh=mesh,
        scratch_types=[
            pltpu.VMEM((BPT, SEQ), jnp.int32),
            pltpu.VMEM((2, SEQ, DIM), jnp.float32),
            pltpu.VMEM((BPT, DIM), jnp.float32),
            pltpu.SemaphoreType.DMA((2,)),
        ],
    )
    return run(instruction.astype(jnp.int32), table)


# speedup vs baseline: 2.8232x; 1.2292x over previous
import jax
import jax.numpy as jnp
from jax import lax
from jax.experimental import pallas as pl
from jax.experimental.pallas import tpu as pltpu
from jax.experimental.pallas import tpu_sc as plsc

BATCH = 4096
SEQ = 200
DIM = 128
NCORES = 2
NSUB = 16
NW = NCORES * NSUB
BPT = BATCH // NW
CH0 = 128
CH1 = SEQ - CH0
NVR = DIM // 16
NBUF = 3


UNROLL = 8


def _body(instr_hbm, table_hbm, out_hbm, idx_v, rows_v, out_v, sems):
    c = lax.axis_index("c")
    s = lax.axis_index("s")
    wid = s * NCORES + c
    base = wid * BPT

    pltpu.sync_copy(instr_hbm.at[pl.ds(base, BPT), :], idx_v)

    def start_gather(b, buf):
        pltpu.async_copy(
            table_hbm.at[idx_v.at[b, pl.ds(0, CH0)]],
            rows_v.at[buf, pl.ds(0, CH0), :], sems.at[buf])
        pltpu.async_copy(
            table_hbm.at[idx_v.at[b, pl.ds(CH0, CH1)]],
            rows_v.at[buf, pl.ds(CH0, CH1), :], sems.at[buf])

    def wait_gather(buf):
        pltpu.make_async_copy(
            table_hbm.at[pl.ds(0, CH0), :],
            rows_v.at[buf, pl.ds(0, CH0), :], sems.at[buf]).wait()
        pltpu.make_async_copy(
            table_hbm.at[pl.ds(0, CH1), :],
            rows_v.at[buf, pl.ds(CH0, CH1), :], sems.at[buf]).wait()

    start_gather(0, 0)
    start_gather(1, 1)

    def bag(b, carry):
        p = lax.rem(b, NBUF)

        @pl.when(b + 2 < BPT)
        def _():
            start_gather(b + 2, lax.rem(b + 2, NBUF))

        wait_gather(p)

        def red(r4, acc):
            r = r4 * UNROLL
            out = []
            for d in range(NVR):
                vals = [rows_v[p, r + k, pl.ds(d * 16, 16)]
                        for k in range(UNROLL)]
                while len(vals) > 1:
                    vals = [vals[i] + vals[i + 1]
                            for i in range(0, len(vals), 2)]
                out.append(acc[d] + vals[0])
            return tuple(out)

        acc = lax.fori_loop(
            0, SEQ // UNROLL, red,
            tuple(jnp.zeros((16,), jnp.float32) for _ in range(NVR)))
        for d in range(NVR):
            out_v[b, pl.ds(d * 16, 16)] = acc[d]
        return carry

    lax.fori_loop(0, BPT, bag, 0)
    pltpu.sync_copy(out_v, out_hbm.at[pl.ds(base, BPT), :])


def kernel(instruction, table):
    mesh = plsc.VectorSubcoreMesh(
        core_axis_name="c", subcore_axis_name="s",
        num_cores=NCORES, num_subcores=NSUB)
    run = pl.kernel(
        _body,
        out_type=jax.ShapeDtypeStruct((BATCH, DIM), jnp.float32),
        mesh=mesh,
        scratch_types=[
            pltpu.VMEM((BPT, SEQ), jnp.int32),
            pltpu.VMEM((NBUF, SEQ, DIM), jnp.float32),
            pltpu.VMEM((BPT, DIM), jnp.float32),
            pltpu.SemaphoreType.DMA((NBUF,)),
        ],
    )
    return run(instruction.astype(jnp.int32), table)


# speedup vs baseline: 2.8350x; 1.0042x over previous
import jax
import jax.numpy as jnp
from jax import lax
from jax.experimental import pallas as pl
from jax.experimental.pallas import tpu as pltpu
from jax.experimental.pallas import tpu_sc as plsc

BATCH = 4096
SEQ = 200
DIM = 128
NCORES = 2
NSUB = 16
NW = NCORES * NSUB
BPT = BATCH // NW
CH0 = 128
CH1 = SEQ - CH0
NVR = DIM // 16
NBUF = 4
UNROLL = 8


def _body(instr_hbm, table_hbm, out_hbm, idx_v, rows_v, stage_v,
          isems, sems, osems):
    c = lax.axis_index("c")
    s = lax.axis_index("s")
    wid = s * NCORES + c
    base = wid * BPT

    def start_idx(b, slot):
        pltpu.async_copy(instr_hbm.at[base + b, :], idx_v.at[slot],
                         isems.at[slot])

    def wait_idx(slot):
        pltpu.make_async_copy(instr_hbm.at[0, :], idx_v.at[slot],
                              isems.at[slot]).wait()

    def start_gather(slot, buf):
        pltpu.async_copy(
            table_hbm.at[idx_v.at[slot, pl.ds(0, CH0)]],
            rows_v.at[buf, pl.ds(0, CH0), :], sems.at[buf])
        pltpu.async_copy(
            table_hbm.at[idx_v.at[slot, pl.ds(CH0, CH1)]],
            rows_v.at[buf, pl.ds(CH0, CH1), :], sems.at[buf])

    def wait_gather(buf):
        pltpu.make_async_copy(
            table_hbm.at[pl.ds(0, CH0), :],
            rows_v.at[buf, pl.ds(0, CH0), :], sems.at[buf]).wait()
        pltpu.make_async_copy(
            table_hbm.at[pl.ds(0, CH1), :],
            rows_v.at[buf, pl.ds(CH0, CH1), :], sems.at[buf]).wait()

    def wait_out(slot):
        pltpu.make_async_copy(
            stage_v.at[slot], out_hbm.at[0, :], osems.at[slot]).wait()

    for j in range(NBUF):
        start_idx(j, j)
    for j in range(NBUF - 1):
        wait_idx(j)
        start_gather(j, j)

    def bag(b, carry):
        p = lax.rem(b, NBUF)
        q = lax.rem(b, 2)

        @pl.when(b + NBUF - 1 < BPT)
        def _():
            slot = lax.rem(b + NBUF - 1, NBUF)
            wait_idx(slot)
            start_gather(slot, slot)

        wait_gather(p)

        @pl.when(b + NBUF < BPT)
        def _():
            start_idx(b + NBUF, p)

        def red(r4, acc):
            r = r4 * UNROLL
            out = []
            for d in range(NVR):
                vals = [rows_v[p, r + k, pl.ds(d * 16, 16)]
                        for k in range(UNROLL)]
                while len(vals) > 1:
                    vals = [vals[i] + vals[i + 1]
                            for i in range(0, len(vals), 2)]
                out.append(acc[d] + vals[0])
            return tuple(out)

        acc = lax.fori_loop(
            0, SEQ // UNROLL, red,
            tuple(jnp.zeros((16,), jnp.float32) for _ in range(NVR)))

        @pl.when(b >= 2)
        def _():
            wait_out(q)

        for d in range(NVR):
            stage_v[q, pl.ds(d * 16, 16)] = acc[d]
        pltpu.async_copy(stage_v.at[q], out_hbm.at[base + b, :],
                         osems.at[q])
        return carry

    lax.fori_loop(0, BPT, bag, 0)
    wait_out(0)
    wait_out(1)


def kernel(instruction, table):
    mesh = plsc.VectorSubcoreMesh(
        core_axis_name="c", subcore_axis_name="s",
        num_cores=NCORES, num_subcores=NSUB)
    run = pl.kernel(
        _body,
        out_type=jax.ShapeDtypeStruct((BATCH, DIM), jnp.float32),
        mesh=mesh,
        scratch_types=[
            pltpu.VMEM((NBUF, SEQ), jnp.int32),
            pltpu.VMEM((NBUF, SEQ, DIM), jnp.float32),
            pltpu.VMEM((2, DIM), jnp.float32),
            pltpu.SemaphoreType.DMA((NBUF,)),
            pltpu.SemaphoreType.DMA((NBUF,)),
            pltpu.SemaphoreType.DMA((2,)),
        ],
    )
    return run(instruction.astype(jnp.int32), table)
